# baseline (speedup 1.0000x reference)
import functools

import jax
import jax.numpy as jnp
from jax import lax
from jax.experimental import pallas as pl
from jax.experimental.pallas import tpu as pltpu
from jax.experimental.pallas import tpu_sc as plsc

B = 16384
V = 100001
EMB = 64
NFEAT = 4
OUT_D = NFEAT * EMB

_info = plsc.get_sparse_core_info()
_NC, _NS = _info.num_cores, _info.num_subcores
_NW = _NC * _NS
_DPW = NFEAT * EMB // _NW
_Q = B // 4


def _body(w0, w1, w2, w3, xT, out, idx_v, slab, obuf, osem0, osem1):
    wid = lax.axis_index("s") * _NC + lax.axis_index("c")
    f = wid // _DPW
    pltpu.sync_copy(xT.at[pl.ds(f, 1)], idx_v)

    tables = (w0, w1, w2, w3)
    zeros = jnp.zeros((16,), jnp.int32)
    osems = (osem0, osem1)
    writes = []
    for ei in range(_DPW):
        e = (wid % _DPW) * _DPW + ei
        for fi in range(NFEAT):
            @pl.when(f == fi)
            def _():
                pltpu.sync_copy(tables[fi].at[pl.ds(e, 1)], slab)
        for h in range(4):
            @plsc.parallel_loop(0, _Q // 16, unroll=16)
            def grp(g):
                iv = idx_v[0, pl.ds(h * _Q + g * 16, 16)]
                vals = plsc.load_gather(slab, [zeros, iv])
                obuf[h % 2, 0, pl.ds(g * 16, 16)] = vals
            if len(writes) >= 2:
                writes.pop(0).wait()
            writes.append(pltpu.async_copy(
                obuf.at[h % 2],
                out.at[pl.ds(f * EMB + e, 1), pl.ds(h * _Q, _Q)],
                osems[h % 2]))
    for wcp in writes:
        wcp.wait()


@jax.jit
def _encode(w0, w1, w2, w3, xT):
    mesh = plsc.VectorSubcoreMesh(core_axis_name="c", subcore_axis_name="s")
    k = functools.partial(
        pl.kernel,
        mesh=mesh,
        compiler_params=pltpu.CompilerParams(needs_layout_passes=False),
        out_type=jax.ShapeDtypeStruct((OUT_D, B), jnp.float32),
        scratch_types=[
            pltpu.VMEM((1, B), jnp.int32),
            pltpu.VMEM((1, V), jnp.float32),
            pltpu.VMEM((2, 1, _Q), jnp.float32),
            pltpu.SemaphoreType.DMA,
            pltpu.SemaphoreType.DMA,
        ],
    )(_body)
    return k(w0, w1, w2, w3, xT)


def kernel(X_cat, W_zipcode, W_category, W_brand, W_platform):
    outT = _encode(W_zipcode.T, W_category.T, W_brand.T, W_platform.T,
                   X_cat.T)
    return outT.T


# speedup vs baseline: 1.0209x; 1.0209x over previous
import functools

import jax
import jax.numpy as jnp
from jax import lax
from jax.experimental import pallas as pl
from jax.experimental.pallas import tpu as pltpu
from jax.experimental.pallas import tpu_sc as plsc

B = 16384
V = 100001
EMB = 64
NFEAT = 4
OUT_D = NFEAT * EMB

_info = plsc.get_sparse_core_info()
_NC, _NS = _info.num_cores, _info.num_subcores
_NW = _NC * _NS
_DPW = NFEAT * EMB // _NW
_Q = B // 4


def _body(w0, w1, w2, w3, xT, out, idx_v, slab, obuf, osem0, osem1):
    wid = lax.axis_index("s") * _NC + lax.axis_index("c")
    f = wid // _DPW
    pltpu.sync_copy(xT.at[pl.ds(f, 1)], idx_v)

    tables = (w0, w1, w2, w3)
    zeros = jnp.zeros((16,), jnp.int32)
    osems = (osem0, osem1)
    writes = []
    for ei in range(_DPW):
        e = (wid % _DPW) * _DPW + ei
        for fi in range(NFEAT):
            @pl.when(f == fi)
            def _():
                pltpu.sync_copy(tables[fi].at[pl.ds(e, 1)], slab)
        for h in range(4):
            @plsc.parallel_loop(0, _Q // 16, unroll=8)
            def grp(g):
                iv = idx_v[0, pl.ds(h * _Q + g * 16, 16)]
                vals = plsc.load_gather(slab, [zeros, iv])
                obuf[h % 2, 0, pl.ds(g * 16, 16)] = vals
            if len(writes) >= 2:
                writes.pop(0).wait()
            writes.append(pltpu.async_copy(
                obuf.at[h % 2],
                out.at[pl.ds(f * EMB + e, 1), pl.ds(h * _Q, _Q)],
                osems[h % 2]))
    for wcp in writes:
        wcp.wait()


@jax.jit
def _encode(w0, w1, w2, w3, xT):
    mesh = plsc.VectorSubcoreMesh(core_axis_name="c", subcore_axis_name="s")
    k = functools.partial(
        pl.kernel,
        mesh=mesh,
        compiler_params=pltpu.CompilerParams(needs_layout_passes=False),
        out_type=jax.ShapeDtypeStruct((OUT_D, B), jnp.float32),
        scratch_types=[
            pltpu.VMEM((1, B), jnp.int32),
            pltpu.VMEM((1, V), jnp.float32),
            pltpu.VMEM((2, 1, _Q), jnp.float32),
            pltpu.SemaphoreType.DMA,
            pltpu.SemaphoreType.DMA,
        ],
    )(_body)
    return k(w0, w1, w2, w3, xT)


def kernel(X_cat, W_zipcode, W_category, W_brand, W_platform):
    outT = _encode(W_zipcode.T, W_category.T, W_brand.T, W_platform.T,
                   X_cat.T)
    return outT.T
